# Initial kernel scaffold; baseline (speedup 1.0000x reference)
import functools

import jax
import jax.numpy as jnp
from jax import lax
from jax.experimental import pallas as pl
from jax.experimental.pallas import tpu as pltpu
from jax.experimental.pallas import tpu_sc as plsc

N_NODES = 10000
N_EDGES = 320000
D = 128
G_POOL = 64

NC = 2
NS = 16
NW = NC * NS
EPW = N_EDGES // NW
CHUNK = 80
NCHUNK = EPW // CHUNK
ROWS_PT = N_NODES // NS
ZROWS = 125


def _sc_aggregate(compute_deg: bool):
  mesh = plsc.VectorSubcoreMesh(core_axis_name="c", subcore_axis_name="s")
  out_type = [jax.ShapeDtypeStruct((NC, N_NODES, D), jnp.float32)]
  if compute_deg:
    out_type.append(jax.ShapeDtypeStruct((NW, N_NODES), jnp.float32))

  scratch = [
      pltpu.VMEM((CHUNK,), jnp.int32),
      pltpu.VMEM((CHUNK,), jnp.int32),
      pltpu.VMEM((CHUNK,), jnp.float32),
      pltpu.VMEM((CHUNK, D), jnp.float32),
      pltpu.VMEM((ZROWS, D), jnp.float32),
      pltpu.VMEM_SHARED((N_NODES, D), jnp.float32),
      pltpu.SemaphoreType.DMA,
  ]
  if compute_deg:
    scratch.append(pltpu.VMEM((N_NODES,), jnp.float32))

  @functools.partial(pl.kernel, mesh=mesh, out_type=tuple(out_type),
                     scratch_types=tuple(scratch))
  def kern(x_hbm, src_hbm, dst_hbm, w_hbm, *refs):
    if compute_deg:
      (parts_hbm, degp_hbm, src_v, dst_v, w_v, rows, zbuf, agg_sh, sem,
       deg_t) = refs
    else:
      parts_hbm, src_v, dst_v, w_v, rows, zbuf, agg_sh, sem = refs
      degp_hbm = deg_t = None

    cid = lax.axis_index("c")
    sid = lax.axis_index("s")
    wid = sid * NC + cid

    zero16 = jnp.zeros((16,), jnp.float32)

    def zrow(k, _):
      for j in range(D // 16):
        zbuf[k, pl.ds(j * 16, 16)] = zero16
      return 0
    lax.fori_loop(0, ZROWS, zrow, 0)
    row0 = sid * ROWS_PT
    for c in range(ROWS_PT // ZROWS):
      pltpu.sync_copy(zbuf, agg_sh.at[pl.ds(row0 + c * ZROWS, ZROWS)])

    if compute_deg:
      def zdeg(k, _):
        deg_t[pl.ds(k * 16, 16)] = zero16
        return 0
      lax.fori_loop(0, N_NODES // 16, zdeg, 0)

    plsc.subcore_barrier()

    ebase = wid * EPW

    def chunk_body(i, _):
      base = ebase + i * CHUNK
      pltpu.sync_copy(src_hbm.at[pl.ds(base, CHUNK)], src_v)
      pltpu.sync_copy(dst_hbm.at[pl.ds(base, CHUNK)], dst_v)
      pltpu.sync_copy(w_hbm.at[pl.ds(base, CHUNK)], w_v)
      pltpu.async_copy(x_hbm.at[src_v], rows, sem).wait()

      def scale_row(k, _):
        wk = plsc.load_gather(w_v, [jnp.full((16,), 0, jnp.int32) + k])
        for j in range(D // 16):
          rows[k, pl.ds(j * 16, 16)] = rows[k, pl.ds(j * 16, 16)] * wk
        return 0
      lax.fori_loop(0, CHUNK, scale_row, 0)

      if compute_deg:
        def deg_step(t, _):
          d16 = dst_v[pl.ds(t * 16, 16)]
          w16 = w_v[pl.ds(t * 16, 16)]
          plsc.addupdate_scatter(deg_t, [d16], w16)
          return 0
        lax.fori_loop(0, CHUNK // 16, deg_step, 0)

      pltpu.sync_copy(rows, agg_sh.at[dst_v], add=True)
      return 0

    lax.fori_loop(0, NCHUNK, chunk_body, 0)

    plsc.subcore_barrier()

    pltpu.sync_copy(agg_sh.at[pl.ds(row0, ROWS_PT)],
                    parts_hbm.at[cid, pl.ds(row0, ROWS_PT)])
    if compute_deg:
      pltpu.sync_copy(deg_t, degp_hbm.at[wid])

  return kern


_sc_agg_deg = _sc_aggregate(True)
_sc_agg = _sc_aggregate(False)


ROW_T = 1000
GRID = N_NODES // ROW_T


def _tc_block_body(parts_ref, degp_ref, x_ref, w_ref, b_ref, o_ref, *,
                   residual):
  deg = jnp.sum(degp_ref[...], axis=0)
  inv = 1.0 / jnp.clip(deg, 1e-6, None)
  a = (parts_ref[0] + parts_ref[1]) * inv[:, None]
  h = jnp.dot(a, w_ref[...], preferred_element_type=jnp.float32) + b_ref[...]
  if residual:
    h = h + x_ref[...]
  o_ref[...] = jnp.maximum(h, 0.0)


def _tc_block(parts, degp, x, w, b, residual):
  body = functools.partial(_tc_block_body, residual=residual)
  return pl.pallas_call(
      body,
      grid=(GRID,),
      in_specs=[
          pl.BlockSpec((NC, ROW_T, D), lambda i: (0, i, 0)),
          pl.BlockSpec((NW, ROW_T), lambda i: (0, i)),
          pl.BlockSpec((ROW_T, D), lambda i: (i, 0)),
          pl.BlockSpec((D, D), lambda i: (0, 0)),
          pl.BlockSpec((1, D), lambda i: (0, 0)),
      ],
      out_specs=pl.BlockSpec((ROW_T, D), lambda i: (i, 0)),
      out_shape=jax.ShapeDtypeStruct((N_NODES, D), jnp.float32),
  )(parts, degp, x, w, b)


def _tc_pool_head_body(h_ref, batch_ref, wdi, bdi, wd0, bd0, wd1, bd1, wout,
                       bout, o_ref, acc, cnt):
  i = pl.program_id(0)

  @pl.when(i == 0)
  def _():
    acc[...] = jnp.zeros_like(acc)
    cnt[...] = jnp.zeros_like(cnt)

  b = batch_ref[0, 0, :]
  seg = lax.broadcasted_iota(jnp.int32, (G_POOL, ROW_T), 0)
  mask = (seg == b[None, :]).astype(jnp.float32)
  acc[...] += jnp.dot(mask, h_ref[...], preferred_element_type=jnp.float32)
  cnt[...] += jnp.sum(mask, axis=1)[:, None]

  @pl.when(i == GRID - 1)
  def _():
    flat = acc[...] / jnp.clip(cnt[...], 1.0, None)
    h1 = jnp.maximum(jnp.dot(flat, wdi[...],
                             preferred_element_type=jnp.float32) + bdi[...], 0.)
    h2 = jnp.maximum(jnp.dot(h1, wd0[...],
                             preferred_element_type=jnp.float32) + bd0[...], 0.)
    h3 = jnp.maximum(jnp.dot(h2, wd1[...],
                             preferred_element_type=jnp.float32) + bd1[...], 0.)
    o_ref[...] = jnp.dot(h3, wout[...],
                         preferred_element_type=jnp.float32) + bout[...]


def _tc_pool_head(h, batch3, wdi, bdi, wd0, bd0, wd1, bd1, wout_p, bout_p):
  wspec = pl.BlockSpec((D, D), lambda i: (0, 0))
  bspec = pl.BlockSpec((1, D), lambda i: (0, 0))
  return pl.pallas_call(
      _tc_pool_head_body,
      grid=(GRID,),
      in_specs=[
          pl.BlockSpec((ROW_T, D), lambda i: (i, 0)),
          pl.BlockSpec((1, 1, ROW_T), lambda i: (i, 0, 0)),
          wspec, bspec, wspec, bspec, wspec, bspec, wspec, bspec,
      ],
      out_specs=pl.BlockSpec((G_POOL, D), lambda i: (0, 0)),
      out_shape=jax.ShapeDtypeStruct((G_POOL, D), jnp.float32),
      scratch_shapes=[
          pltpu.VMEM((G_POOL, D), jnp.float32),
          pltpu.VMEM((G_POOL, D), jnp.float32),
      ],
      compiler_params=pltpu.CompilerParams(
          dimension_semantics=("arbitrary",)),
  )(h, batch3, wdi, bdi, wd0, bd0, wd1, bd1, wout_p, bout_p)


def kernel(inputs, edge_index, batch, edge_weight, Win, bin0, W1, b1, W2, b2,
           Wdi, bdi, Wd0, bd0, Wd1, bd1, Wout, bout):
  src = edge_index[0]
  dst = edge_index[1]

  parts1, degp = _sc_agg_deg(inputs, src, dst, edge_weight)
  h1 = _tc_block(parts1, degp, inputs, Win, bin0.reshape(1, D), False)

  (parts2,) = _sc_agg(h1, src, dst, edge_weight)
  h2 = _tc_block(parts2, degp, h1, W1, b1.reshape(1, D), True)

  (parts3,) = _sc_agg(h2, src, dst, edge_weight)
  h3 = _tc_block(parts3, degp, h2, W2, b2.reshape(1, D), True)

  batch3 = batch.reshape(GRID, 1, ROW_T)
  wout_p = jnp.zeros((D, D), jnp.float32).at[:, :Wout.shape[1]].set(Wout)
  bout_p = jnp.zeros((1, D), jnp.float32).at[0, :bout.shape[0]].set(bout)
  out = _tc_pool_head(h3, batch3, Wdi, bdi.reshape(1, D), Wd0,
                      bd0.reshape(1, D), Wd1, bd1.reshape(1, D),
                      wout_p, bout_p)
  return out[:, :Wout.shape[1]]


# speedup vs baseline: 4.0802x; 4.0802x over previous
import functools

import jax
import jax.numpy as jnp
from jax import lax
from jax.experimental import pallas as pl
from jax.experimental.pallas import tpu as pltpu
from jax.experimental.pallas import tpu_sc as plsc

N_NODES = 10000
N_PAD = 10240
N_EDGES = 320000
D = 128
G_POOL = 64

NC = 2
NS = 16
NW = NC * NS
EPW = N_EDGES // NW
CHUNK = 80
NCHUNK = EPW // CHUNK
ROWS_PT = N_PAD // NS
ZROWS = 128


def _sc_aggregate(compute_deg: bool):
  mesh = plsc.VectorSubcoreMesh(core_axis_name="c", subcore_axis_name="s")
  out_type = [jax.ShapeDtypeStruct((NC, N_PAD, D), jnp.float32)]
  if compute_deg:
    out_type.append(jax.ShapeDtypeStruct((NW, N_PAD), jnp.float32))

  scratch = [
      pltpu.VMEM((CHUNK,), jnp.int32),
      pltpu.VMEM((CHUNK,), jnp.int32),
      pltpu.VMEM((CHUNK,), jnp.float32),
      pltpu.VMEM((CHUNK, D), jnp.float32),
      pltpu.VMEM((ZROWS, D), jnp.float32),
      pltpu.VMEM_SHARED((N_PAD, D), jnp.float32),
      pltpu.SemaphoreType.DMA,
  ]
  if compute_deg:
    scratch.append(pltpu.VMEM((N_PAD,), jnp.float32))

  @functools.partial(pl.kernel, mesh=mesh, out_type=tuple(out_type),
                     scratch_types=tuple(scratch),
                     compiler_params=pltpu.CompilerParams(
                         needs_layout_passes=False))
  def kern(x_hbm, src_hbm, dst_hbm, w_hbm, *refs):
    if compute_deg:
      (parts_hbm, degp_hbm, src_v, dst_v, w_v, rows, zbuf, agg_sh, sem,
       deg_t) = refs
    else:
      parts_hbm, src_v, dst_v, w_v, rows, zbuf, agg_sh, sem = refs
      degp_hbm = deg_t = None

    cid = lax.axis_index("c")
    sid = lax.axis_index("s")
    wid = sid * NC + cid

    zero16 = jnp.zeros((16,), jnp.float32)

    def zrow(k, _):
      for j in range(D // 16):
        zbuf[k, pl.ds(j * 16, 16)] = zero16
      return 0
    lax.fori_loop(0, ZROWS, zrow, 0)
    row0 = sid * ROWS_PT
    for c in range(ROWS_PT // ZROWS):
      pltpu.sync_copy(zbuf, agg_sh.at[pl.ds(row0 + c * ZROWS, ZROWS)])

    if compute_deg:
      def zdeg(k, _):
        deg_t[pl.ds(k * 16, 16)] = zero16
        return 0
      lax.fori_loop(0, N_PAD // 16, zdeg, 0)

    plsc.subcore_barrier()

    ebase = wid * EPW

    def chunk_body(i, _):
      base = ebase + i * CHUNK
      pltpu.sync_copy(src_hbm.at[pl.ds(base, CHUNK)], src_v)
      pltpu.sync_copy(dst_hbm.at[pl.ds(base, CHUNK)], dst_v)
      pltpu.sync_copy(w_hbm.at[pl.ds(base, CHUNK)], w_v)
      pltpu.async_copy(x_hbm.at[src_v], rows, sem).wait()

      def scale_group(g, _):
        w16 = w_v[pl.ds(g * 16, 16)]
        for r in range(16):
          wk = jnp.full((16,), w16[r], jnp.float32)
          k = g * 16 + r
          for j in range(D // 16):
            rows[k, pl.ds(j * 16, 16)] = rows[k, pl.ds(j * 16, 16)] * wk
        return 0
      lax.fori_loop(0, CHUNK // 16, scale_group, 0)

      if compute_deg:
        def deg_step(t, _):
          d16 = dst_v[pl.ds(t * 16, 16)]
          w16 = w_v[pl.ds(t * 16, 16)]
          plsc.addupdate_scatter(deg_t, [d16], w16)
          return 0
        lax.fori_loop(0, CHUNK // 16, deg_step, 0)

      pltpu.sync_copy(rows, agg_sh.at[dst_v], add=True)
      return 0

    lax.fori_loop(0, NCHUNK, chunk_body, 0)

    plsc.subcore_barrier()

    pltpu.sync_copy(agg_sh.at[pl.ds(row0, ROWS_PT)],
                    parts_hbm.at[cid, pl.ds(row0, ROWS_PT)])
    if compute_deg:
      pltpu.sync_copy(deg_t, degp_hbm.at[wid])

  return kern


_sc_agg_deg = _sc_aggregate(True)
_sc_agg = _sc_aggregate(False)


ROW_T = 1024
GRID = N_PAD // ROW_T


def _tc_block_body(parts_ref, degp_ref, x_ref, w_ref, b_ref, o_ref, *,
                   residual):
  deg = jnp.sum(degp_ref[...], axis=0)
  inv = 1.0 / jnp.clip(deg, 1e-6, None)
  a = (parts_ref[0] + parts_ref[1]) * inv[:, None]
  h = jnp.dot(a, w_ref[...], preferred_element_type=jnp.float32) + b_ref[...]
  if residual:
    h = h + x_ref[...]
  o_ref[...] = jnp.maximum(h, 0.0)


def _tc_block(parts, degp, x, w, b, residual):
  body = functools.partial(_tc_block_body, residual=residual)
  return pl.pallas_call(
      body,
      grid=(GRID,),
      in_specs=[
          pl.BlockSpec((NC, ROW_T, D), lambda i: (0, i, 0)),
          pl.BlockSpec((NW, ROW_T), lambda i: (0, i)),
          pl.BlockSpec((ROW_T, D), lambda i: (i, 0)),
          pl.BlockSpec((D, D), lambda i: (0, 0)),
          pl.BlockSpec((1, D), lambda i: (0, 0)),
      ],
      out_specs=pl.BlockSpec((ROW_T, D), lambda i: (i, 0)),
      out_shape=jax.ShapeDtypeStruct((N_PAD, D), jnp.float32),
  )(parts, degp, x, w, b)


def _tc_pool_head_body(h_ref, batch_ref, wdi, bdi, wd0, bd0, wd1, bd1, wout,
                       bout, o_ref, acc, cnt):
  i = pl.program_id(0)

  @pl.when(i == 0)
  def _():
    acc[...] = jnp.zeros_like(acc)
    cnt[...] = jnp.zeros_like(cnt)

  b = batch_ref[0, 0, :]
  seg = lax.broadcasted_iota(jnp.int32, (G_POOL, ROW_T), 0)
  mask = (seg == b[None, :]).astype(jnp.float32)
  acc[...] += jnp.dot(mask, h_ref[...], preferred_element_type=jnp.float32)
  cnt[...] += jnp.dot(mask, jnp.ones((ROW_T, D), jnp.float32),
                      preferred_element_type=jnp.float32)

  @pl.when(i == GRID - 1)
  def _():
    flat = acc[...] / jnp.clip(cnt[...], 1.0, None)
    h1 = jnp.maximum(jnp.dot(flat, wdi[...],
                             preferred_element_type=jnp.float32) + bdi[...], 0.)
    h2 = jnp.maximum(jnp.dot(h1, wd0[...],
                             preferred_element_type=jnp.float32) + bd0[...], 0.)
    h3 = jnp.maximum(jnp.dot(h2, wd1[...],
                             preferred_element_type=jnp.float32) + bd1[...], 0.)
    o_ref[...] = jnp.dot(h3, wout[...],
                         preferred_element_type=jnp.float32) + bout[...]


def _tc_pool_head(h, batch3, wdi, bdi, wd0, bd0, wd1, bd1, wout_p, bout_p):
  wspec = pl.BlockSpec((D, D), lambda i: (0, 0))
  bspec = pl.BlockSpec((1, D), lambda i: (0, 0))
  return pl.pallas_call(
      _tc_pool_head_body,
      grid=(GRID,),
      in_specs=[
          pl.BlockSpec((ROW_T, D), lambda i: (i, 0)),
          pl.BlockSpec((1, 1, ROW_T), lambda i: (i, 0, 0)),
          wspec, bspec, wspec, bspec, wspec, bspec, wspec, bspec,
      ],
      out_specs=pl.BlockSpec((G_POOL, D), lambda i: (0, 0)),
      out_shape=jax.ShapeDtypeStruct((G_POOL, D), jnp.float32),
      scratch_shapes=[
          pltpu.VMEM((G_POOL, D), jnp.float32),
          pltpu.VMEM((G_POOL, D), jnp.float32),
      ],
      compiler_params=pltpu.CompilerParams(
          dimension_semantics=("arbitrary",)),
  )(h, batch3, wdi, bdi, wd0, bd0, wd1, bd1, wout_p, bout_p)


def kernel(inputs, edge_index, batch, edge_weight, Win, bin0, W1, b1, W2, b2,
           Wdi, bdi, Wd0, bd0, Wd1, bd1, Wout, bout):
  src = edge_index[0]
  dst = edge_index[1]

  x0 = jnp.zeros((N_PAD, D), jnp.float32).at[:N_NODES].set(inputs)

  parts1, degp = _sc_agg_deg(x0, src, dst, edge_weight)
  h1 = _tc_block(parts1, degp, x0, Win, bin0.reshape(1, D), False)

  (parts2,) = _sc_agg(h1, src, dst, edge_weight)
  h2 = _tc_block(parts2, degp, h1, W1, b1.reshape(1, D), True)

  (parts3,) = _sc_agg(h2, src, dst, edge_weight)
  h3 = _tc_block(parts3, degp, h2, W2, b2.reshape(1, D), True)

  batch_p = jnp.full((N_PAD,), G_POOL, jnp.int32).at[:N_NODES].set(batch)
  batch3 = batch_p.reshape(GRID, 1, ROW_T)
  wout_p = jnp.zeros((D, D), jnp.float32).at[:, :Wout.shape[1]].set(Wout)
  bout_p = jnp.zeros((1, D), jnp.float32).at[0, :bout.shape[0]].set(bout)
  out = _tc_pool_head(h3, batch3, Wdi, bdi.reshape(1, D), Wd0,
                      bd0.reshape(1, D), Wd1, bd1.reshape(1, D),
                      wout_p, bout_p)
  return out[:, :Wout.shape[1]]


# speedup vs baseline: 12.0644x; 2.9568x over previous
import functools

import jax
import jax.numpy as jnp
from jax import lax
from jax.experimental import pallas as pl
from jax.experimental.pallas import tpu as pltpu
from jax.experimental.pallas import tpu_sc as plsc

N_NODES = 10000
N_PAD = 10240
N_EDGES = 320000
D = 128
G_POOL = 64

NC = 2
NS = 16
NW = NC * NS
EPW = N_EDGES // NW
CHUNK = 80
NCHUNK = EPW // CHUNK
ROWS_PT = N_PAD // NS


def _sc_aggregate(compute_deg: bool, nbuf: int):
  pref = nbuf - 1
  mesh = plsc.VectorSubcoreMesh(core_axis_name="c", subcore_axis_name="s")
  out_type = [jax.ShapeDtypeStruct((NC, N_PAD, D), jnp.float32)]
  if compute_deg:
    out_type.append(jax.ShapeDtypeStruct((NW, N_PAD), jnp.float32))

  scratch = [
      pltpu.VMEM((nbuf, CHUNK), jnp.int32),
      pltpu.VMEM((nbuf, CHUNK), jnp.int32),
      pltpu.VMEM((nbuf, CHUNK), jnp.float32),
  ]
  scratch += [pltpu.VMEM((CHUNK, D), jnp.float32) for _ in range(nbuf)]
  scratch += [pltpu.VMEM((CHUNK,), jnp.int32) for _ in range(nbuf)]
  scratch += [pltpu.VMEM_SHARED((N_PAD, D), jnp.float32)]
  scratch += [pltpu.SemaphoreType.DMA] * (3 * nbuf)
  if compute_deg:
    scratch.append(pltpu.VMEM((N_PAD,), jnp.float32))

  @functools.partial(pl.kernel, mesh=mesh, out_type=tuple(out_type),
                     scratch_types=tuple(scratch),
                     compiler_params=pltpu.CompilerParams(
                         needs_layout_passes=False))
  def kern(x_hbm, src_hbm, dst_hbm, w_hbm, *refs):
    no = 2 if compute_deg else 1
    parts_hbm = refs[0]
    degp_hbm = refs[1] if compute_deg else None
    esrc, edst, ew = refs[no:no + 3]
    rows = refs[no + 3:no + 3 + nbuf]
    sdst = refs[no + 3 + nbuf:no + 3 + 2 * nbuf]
    agg_sh = refs[no + 3 + 2 * nbuf]
    isem = refs[no + 4 + 2 * nbuf:no + 4 + 3 * nbuf]
    gsem = refs[no + 4 + 3 * nbuf:no + 4 + 4 * nbuf]
    ssem = refs[no + 4 + 4 * nbuf:no + 4 + 5 * nbuf]
    deg_t = refs[-1] if compute_deg else None

    cid = lax.axis_index("c")
    sid = lax.axis_index("s")
    wid = sid * NC + cid

    zero16 = jnp.zeros((16,), jnp.float32)

    def zrow(k, _):
      for j in range(D // 16):
        rows[0][k, pl.ds(j * 16, 16)] = zero16
      return 0
    lax.fori_loop(0, CHUNK, zrow, 0)
    row0 = sid * ROWS_PT
    for c in range(ROWS_PT // CHUNK):
      pltpu.sync_copy(rows[0], agg_sh.at[pl.ds(row0 + c * CHUNK, CHUNK)])

    if compute_deg:
      def zdeg(k, _):
        deg_t[pl.ds(k * 16, 16)] = zero16
        return 0
      lax.fori_loop(0, N_PAD // 16, zdeg, 0)

    plsc.subcore_barrier()

    def start_idx(q, s):
      pltpu.async_copy(src_hbm.at[wid, q], esrc.at[s], isem[s])
      pltpu.async_copy(dst_hbm.at[wid, q], edst.at[s], isem[s])
      pltpu.async_copy(w_hbm.at[wid, q], ew.at[s], isem[s])

    def wait_idx(s):
      pltpu.make_async_copy(src_hbm.at[wid, 0], esrc.at[s], isem[s]).wait()
      pltpu.make_async_copy(dst_hbm.at[wid, 0], edst.at[s], isem[s]).wait()
      pltpu.make_async_copy(w_hbm.at[wid, 0], ew.at[s], isem[s]).wait()

    def start_gather(b):
      pltpu.async_copy(x_hbm.at[esrc.at[b]], rows[b], gsem[b])

    def wait_gather(b):
      pltpu.make_async_copy(x_hbm.at[esrc.at[b]], rows[b], gsem[b]).wait()

    def start_scatter(b):
      pltpu.async_copy(rows[b], agg_sh.at[sdst[b]], ssem[b], add=True)

    def wait_scatter(b):
      pltpu.make_async_copy(rows[b], agg_sh.at[sdst[b]], ssem[b]).wait()

    def chunk_work(b):
      wait_gather(b)

      for g in range(CHUNK // 16):
        sdst[b][pl.ds(g * 16, 16)] = edst[b, pl.ds(g * 16, 16)]

      def scale_group(g, _):
        w16 = ew[b, pl.ds(g * 16, 16)]
        for r in range(16):
          wk = jnp.full((16,), w16[r], jnp.float32)
          k = g * 16 + r
          for jj in range(D // 16):
            rows[b][k, pl.ds(jj * 16, 16)] = (
                rows[b][k, pl.ds(jj * 16, 16)] * wk)
        return 0
      lax.fori_loop(0, CHUNK // 16, scale_group, 0)

      if compute_deg:
        def deg_step(t, _):
          d16 = edst[b, pl.ds(t * 16, 16)]
          w16 = ew[b, pl.ds(t * 16, 16)]
          plsc.addupdate_scatter(deg_t, [d16], w16)
          return 0
        lax.fori_loop(0, CHUNK // 16, deg_step, 0)

      start_scatter(b)

    for q in range(pref + 1):
      start_idx(jnp.int32(q), q % nbuf)
    for q in range(pref):
      wait_idx(q)
      start_gather(q)

    def chunk_full(j, b, g_guard, s_guard, i_guard):
      chunk_work(b)
      gb = (b + pref) % nbuf
      if g_guard:
        if s_guard:
          wait_scatter(gb)
        wait_idx(gb)
        start_gather(gb)
      ib = (b + pref + 1) % nbuf
      if i_guard:
        start_idx(j + pref + 1, ib)

    for i in range(nbuf):
      chunk_full(jnp.int32(i), i, i + pref < NCHUNK,
                 i + pref - nbuf >= 0, i + pref + 1 < NCHUNK)

    nround = NCHUNK // nbuf

    def round_body(t, _):
      for b in range(nbuf):
        j = t * nbuf + b
        chunk_work(b)
        gb = (b + pref) % nbuf

        @pl.when(j + pref < NCHUNK)
        def _():
          wait_scatter(gb)
          wait_idx(gb)
          start_gather(gb)

        ib = (b + pref + 1) % nbuf

        @pl.when(j + pref + 1 < NCHUNK)
        def _():
          start_idx(j + pref + 1, ib)
      return 0
    lax.fori_loop(1, nround, round_body, 0)

    for j in range(nround * nbuf, NCHUNK):
      chunk_full(jnp.int32(j), j % nbuf, j + pref < NCHUNK,
                 True, j + pref + 1 < NCHUNK)

    for b in range(nbuf):
      wait_scatter(b)

    plsc.subcore_barrier()

    pltpu.sync_copy(agg_sh.at[pl.ds(row0, ROWS_PT)],
                    parts_hbm.at[cid, pl.ds(row0, ROWS_PT)])
    if compute_deg:
      pltpu.sync_copy(deg_t, degp_hbm.at[wid])

  return kern


_sc_agg_deg = _sc_aggregate(True, 3)
_sc_agg = _sc_aggregate(False, 4)


ROW_T = 1024
GRID = N_PAD // ROW_T


def _tc_block_body(parts_ref, degp_ref, x_ref, w_ref, b_ref, o_ref, *,
                   residual):
  deg = jnp.sum(degp_ref[...], axis=0)
  inv = 1.0 / jnp.clip(deg, 1e-6, None)
  a = (parts_ref[0] + parts_ref[1]) * inv[:, None]
  h = jnp.dot(a, w_ref[...], preferred_element_type=jnp.float32) + b_ref[...]
  if residual:
    h = h + x_ref[...]
  o_ref[...] = jnp.maximum(h, 0.0)


def _tc_block(parts, degp, x, w, b, residual):
  body = functools.partial(_tc_block_body, residual=residual)
  return pl.pallas_call(
      body,
      grid=(GRID,),
      in_specs=[
          pl.BlockSpec((NC, ROW_T, D), lambda i: (0, i, 0)),
          pl.BlockSpec((NW, ROW_T), lambda i: (0, i)),
          pl.BlockSpec((ROW_T, D), lambda i: (i, 0)),
          pl.BlockSpec((D, D), lambda i: (0, 0)),
          pl.BlockSpec((1, D), lambda i: (0, 0)),
      ],
      out_specs=pl.BlockSpec((ROW_T, D), lambda i: (i, 0)),
      out_shape=jax.ShapeDtypeStruct((N_PAD, D), jnp.float32),
  )(parts, degp, x, w, b)


def _tc_pool_head_body(h_ref, batch_ref, wdi, bdi, wd0, bd0, wd1, bd1, wout,
                       bout, o_ref, acc, cnt):
  i = pl.program_id(0)

  @pl.when(i == 0)
  def _():
    acc[...] = jnp.zeros_like(acc)
    cnt[...] = jnp.zeros_like(cnt)

  b = batch_ref[0, 0, :]
  seg = lax.broadcasted_iota(jnp.int32, (G_POOL, ROW_T), 0)
  mask = (seg == b[None, :]).astype(jnp.float32)
  acc[...] += jnp.dot(mask, h_ref[...], preferred_element_type=jnp.float32)
  cnt[...] += jnp.dot(mask, jnp.ones((ROW_T, D), jnp.float32),
                      preferred_element_type=jnp.float32)

  @pl.when(i == GRID - 1)
  def _():
    flat = acc[...] / jnp.clip(cnt[...], 1.0, None)
    h1 = jnp.maximum(jnp.dot(flat, wdi[...],
                             preferred_element_type=jnp.float32) + bdi[...], 0.)
    h2 = jnp.maximum(jnp.dot(h1, wd0[...],
                             preferred_element_type=jnp.float32) + bd0[...], 0.)
    h3 = jnp.maximum(jnp.dot(h2, wd1[...],
                             preferred_element_type=jnp.float32) + bd1[...], 0.)
    o_ref[...] = jnp.dot(h3, wout[...],
                         preferred_element_type=jnp.float32) + bout[...]


def _tc_pool_head(h, batch3, wdi, bdi, wd0, bd0, wd1, bd1, wout_p, bout_p):
  wspec = pl.BlockSpec((D, D), lambda i: (0, 0))
  bspec = pl.BlockSpec((1, D), lambda i: (0, 0))
  return pl.pallas_call(
      _tc_pool_head_body,
      grid=(GRID,),
      in_specs=[
          pl.BlockSpec((ROW_T, D), lambda i: (i, 0)),
          pl.BlockSpec((1, 1, ROW_T), lambda i: (i, 0, 0)),
          wspec, bspec, wspec, bspec, wspec, bspec, wspec, bspec,
      ],
      out_specs=pl.BlockSpec((G_POOL, D), lambda i: (0, 0)),
      out_shape=jax.ShapeDtypeStruct((G_POOL, D), jnp.float32),
      scratch_shapes=[
          pltpu.VMEM((G_POOL, D), jnp.float32),
          pltpu.VMEM((G_POOL, D), jnp.float32),
      ],
      compiler_params=pltpu.CompilerParams(
          dimension_semantics=("arbitrary",)),
  )(h, batch3, wdi, bdi, wd0, bd0, wd1, bd1, wout_p, bout_p)


def kernel(inputs, edge_index, batch, edge_weight, Win, bin0, W1, b1, W2, b2,
           Wdi, bdi, Wd0, bd0, Wd1, bd1, Wout, bout):
  src = edge_index[0].reshape(NW, NCHUNK, CHUNK)
  dst = edge_index[1].reshape(NW, NCHUNK, CHUNK)
  ew = edge_weight.reshape(NW, NCHUNK, CHUNK)

  x0 = jnp.zeros((N_PAD, D), jnp.float32).at[:N_NODES].set(inputs)

  parts1, degp = _sc_agg_deg(x0, src, dst, ew)
  h1 = _tc_block(parts1, degp, x0, Win, bin0.reshape(1, D), False)

  (parts2,) = _sc_agg(h1, src, dst, ew)
  h2 = _tc_block(parts2, degp, h1, W1, b1.reshape(1, D), True)

  (parts3,) = _sc_agg(h2, src, dst, ew)
  h3 = _tc_block(parts3, degp, h2, W2, b2.reshape(1, D), True)

  batch_p = jnp.full((N_PAD,), G_POOL, jnp.int32).at[:N_NODES].set(batch)
  batch3 = batch_p.reshape(GRID, 1, ROW_T)
  wout_p = jnp.zeros((D, D), jnp.float32).at[:, :Wout.shape[1]].set(Wout)
  bout_p = jnp.zeros((1, D), jnp.float32).at[0, :bout.shape[0]].set(bout)
  out = _tc_pool_head(h3, batch3, Wdi, bdi.reshape(1, D), Wd0,
                      bd0.reshape(1, D), Wd1, bd1.reshape(1, D),
                      wout_p, bout_p)
  return out[:, :Wout.shape[1]]


# speedup vs baseline: 12.4604x; 1.0328x over previous
import functools

import jax
import jax.numpy as jnp
from jax import lax
from jax.experimental import pallas as pl
from jax.experimental.pallas import tpu as pltpu
from jax.experimental.pallas import tpu_sc as plsc

N_NODES = 10000
N_PAD = 10240
N_EDGES = 320000
D = 128
G_POOL = 64

NC = 2
NS = 16
NW = NC * NS
EPW = N_EDGES // NW
CHUNK = 80
NCHUNK = EPW // CHUNK
ROWS_PT = N_PAD // NS


def _sc_aggregate(compute_deg: bool, nbuf: int):
  pref = nbuf - 1
  mesh = plsc.VectorSubcoreMesh(core_axis_name="c", subcore_axis_name="s")
  out_type = [jax.ShapeDtypeStruct((NC, N_PAD, D), jnp.float32)]
  if compute_deg:
    out_type.append(jax.ShapeDtypeStruct((NW, N_PAD), jnp.float32))

  scratch = [
      pltpu.VMEM((nbuf, CHUNK), jnp.int32),
      pltpu.VMEM((nbuf, CHUNK), jnp.int32),
      pltpu.VMEM((nbuf, CHUNK), jnp.float32),
  ]
  scratch += [pltpu.VMEM((CHUNK, D), jnp.float32) for _ in range(nbuf)]
  scratch += [pltpu.VMEM((CHUNK,), jnp.int32) for _ in range(nbuf)]
  scratch += [pltpu.VMEM_SHARED((N_PAD, D), jnp.float32)]
  scratch += [pltpu.SemaphoreType.DMA] * (3 * nbuf)
  if compute_deg:
    scratch.append(pltpu.VMEM((N_PAD,), jnp.float32))

  @functools.partial(pl.kernel, mesh=mesh, out_type=tuple(out_type),
                     scratch_types=tuple(scratch),
                     compiler_params=pltpu.CompilerParams(
                         needs_layout_passes=False))
  def kern(x_hbm, src_hbm, dst_hbm, w_hbm, *refs):
    no = 2 if compute_deg else 1
    parts_hbm = refs[0]
    degp_hbm = refs[1] if compute_deg else None
    esrc, edst, ew = refs[no:no + 3]
    rows = refs[no + 3:no + 3 + nbuf]
    sdst = refs[no + 3 + nbuf:no + 3 + 2 * nbuf]
    agg_sh = refs[no + 3 + 2 * nbuf]
    isem = refs[no + 4 + 2 * nbuf:no + 4 + 3 * nbuf]
    gsem = refs[no + 4 + 3 * nbuf:no + 4 + 4 * nbuf]
    ssem = refs[no + 4 + 4 * nbuf:no + 4 + 5 * nbuf]
    deg_t = refs[-1] if compute_deg else None

    cid = lax.axis_index("c")
    sid = lax.axis_index("s")
    wid = sid * NC + cid

    zero16 = jnp.zeros((16,), jnp.float32)

    def zrow(k, _):
      for j in range(D // 16):
        rows[0][k, pl.ds(j * 16, 16)] = zero16
      return 0
    lax.fori_loop(0, CHUNK, zrow, 0)
    row0 = sid * ROWS_PT
    for c in range(ROWS_PT // CHUNK):
      pltpu.sync_copy(rows[0], agg_sh.at[pl.ds(row0 + c * CHUNK, CHUNK)])

    if compute_deg:
      def zdeg(k, _):
        deg_t[pl.ds(k * 16, 16)] = zero16
        return 0
      lax.fori_loop(0, N_PAD // 16, zdeg, 0)

    plsc.subcore_barrier()

    def start_idx(q, s):
      pltpu.async_copy(src_hbm.at[wid, q], esrc.at[s], isem[s])
      pltpu.async_copy(dst_hbm.at[wid, q], edst.at[s], isem[s])
      pltpu.async_copy(w_hbm.at[wid, q], ew.at[s], isem[s])

    def wait_idx(s):
      pltpu.make_async_copy(src_hbm.at[wid, 0], esrc.at[s], isem[s]).wait()
      pltpu.make_async_copy(dst_hbm.at[wid, 0], edst.at[s], isem[s]).wait()
      pltpu.make_async_copy(w_hbm.at[wid, 0], ew.at[s], isem[s]).wait()

    def start_gather(b):
      pltpu.async_copy(x_hbm.at[esrc.at[b]], rows[b], gsem[b])

    def wait_gather(b):
      pltpu.make_async_copy(x_hbm.at[esrc.at[b]], rows[b], gsem[b]).wait()

    def start_scatter(b):
      pltpu.async_copy(rows[b], agg_sh.at[sdst[b]], ssem[b], add=True)

    def wait_scatter(b):
      pltpu.make_async_copy(rows[b], agg_sh.at[sdst[b]], ssem[b]).wait()

    def chunk_work(b):
      wait_gather(b)

      for g in range(CHUNK // 16):
        sdst[b][pl.ds(g * 16, 16)] = edst[b, pl.ds(g * 16, 16)]

      def scale_group(g, _):
        w16 = ew[b, pl.ds(g * 16, 16)]
        for r in range(16):
          wk = jnp.full((16,), w16[r], jnp.float32)
          k = g * 16 + r
          for jj in range(D // 16):
            rows[b][k, pl.ds(jj * 16, 16)] = (
                rows[b][k, pl.ds(jj * 16, 16)] * wk)
        return 0
      lax.fori_loop(0, CHUNK // 16, scale_group, 0)

      if compute_deg:
        def deg_step(t, _):
          d16 = edst[b, pl.ds(t * 16, 16)]
          w16 = ew[b, pl.ds(t * 16, 16)]
          plsc.addupdate_scatter(deg_t, [d16], w16)
          return 0
        lax.fori_loop(0, CHUNK // 16, deg_step, 0)

      start_scatter(b)

    for q in range(pref + 1):
      start_idx(jnp.int32(q), q % nbuf)
    for q in range(pref):
      wait_idx(q)
      start_gather(q)

    def chunk_full(j, b, g_guard, s_guard, i_guard):
      chunk_work(b)
      gb = (b + pref) % nbuf
      if g_guard:
        if s_guard:
          wait_scatter(gb)
        wait_idx(gb)
        start_gather(gb)
      ib = (b + pref + 1) % nbuf
      if i_guard:
        start_idx(j + pref + 1, ib)

    for i in range(nbuf):
      chunk_full(jnp.int32(i), i, i + pref < NCHUNK,
                 i + pref - nbuf >= 0, i + pref + 1 < NCHUNK)

    nround = NCHUNK // nbuf

    def round_body(t, _):
      for b in range(nbuf):
        j = t * nbuf + b
        chunk_work(b)
        gb = (b + pref) % nbuf

        @pl.when(j + pref < NCHUNK)
        def _():
          wait_scatter(gb)
          wait_idx(gb)
          start_gather(gb)

        ib = (b + pref + 1) % nbuf

        @pl.when(j + pref + 1 < NCHUNK)
        def _():
          start_idx(j + pref + 1, ib)
      return 0
    lax.fori_loop(1, nround, round_body, 0)

    for j in range(nround * nbuf, NCHUNK):
      chunk_full(jnp.int32(j), j % nbuf, j + pref < NCHUNK,
                 True, j + pref + 1 < NCHUNK)

    for b in range(nbuf):
      wait_scatter(b)

    plsc.subcore_barrier()

    pltpu.sync_copy(agg_sh.at[pl.ds(row0, ROWS_PT)],
                    parts_hbm.at[cid, pl.ds(row0, ROWS_PT)])
    if compute_deg:
      pltpu.sync_copy(deg_t, degp_hbm.at[wid])

  return kern


_sc_agg_deg = _sc_aggregate(True, 3)
_sc_agg = _sc_aggregate(False, 4)


ROW_T = 1024
GRID = N_PAD // ROW_T


def _block_h(parts_ref, degp_ref, w_ref, b_ref, x_ref):
  deg = jnp.sum(degp_ref[...], axis=0)
  inv = 1.0 / jnp.clip(deg, 1e-6, None)
  a = (parts_ref[0] + parts_ref[1]) * inv[:, None]
  h = jnp.dot(a, w_ref[...], preferred_element_type=jnp.float32) + b_ref[...]
  if x_ref is not None:
    h = h + x_ref[...]
  return jnp.maximum(h, 0.0)


def _tc_block_res_body(parts_ref, degp_ref, x_ref, w_ref, b_ref, o_ref):
  o_ref[...] = _block_h(parts_ref, degp_ref, w_ref, b_ref, x_ref)


def _tc_block_nores_body(parts_ref, degp_ref, w_ref, b_ref, o_ref):
  o_ref[...] = _block_h(parts_ref, degp_ref, w_ref, b_ref, None)


def _tc_block(parts, degp, x, w, b, residual):
  specs = [
      pl.BlockSpec((NC, ROW_T, D), lambda i: (0, i, 0)),
      pl.BlockSpec((NW, ROW_T), lambda i: (0, i)),
      pl.BlockSpec((ROW_T, D), lambda i: (i, 0)),
      pl.BlockSpec((D, D), lambda i: (0, 0)),
      pl.BlockSpec((1, D), lambda i: (0, 0)),
  ]
  if residual:
    body, args = _tc_block_res_body, (parts, degp, x, w, b)
  else:
    body, args = _tc_block_nores_body, (parts, degp, w, b)
    specs = [specs[0], specs[1], specs[3], specs[4]]
  return pl.pallas_call(
      body,
      grid=(GRID,),
      in_specs=specs,
      out_specs=pl.BlockSpec((ROW_T, D), lambda i: (i, 0)),
      out_shape=jax.ShapeDtypeStruct((N_PAD, D), jnp.float32),
  )(*args)


def _tc_block3_pool_head_body(parts_ref, degp_ref, x_ref, w2, b2, batch_ref,
                              wdi, bdi, wd0, bd0, wd1, bd1, wout, bout,
                              o_ref, acc, cnt):
  i = pl.program_id(0)

  @pl.when(i == 0)
  def _():
    acc[...] = jnp.zeros_like(acc)
    cnt[...] = jnp.zeros_like(cnt)

  h = _block_h(parts_ref, degp_ref, w2, b2, x_ref)

  b = batch_ref[0, 0, :]
  seg = lax.broadcasted_iota(jnp.int32, (G_POOL, ROW_T), 0)
  mask = (seg == b[None, :]).astype(jnp.float32)
  acc[...] += jnp.dot(mask, h, preferred_element_type=jnp.float32)
  cnt[...] += jnp.dot(mask, jnp.ones((ROW_T, D), jnp.float32),
                      preferred_element_type=jnp.float32)

  @pl.when(i == GRID - 1)
  def _():
    flat = acc[...] / jnp.clip(cnt[...], 1.0, None)
    h1 = jnp.maximum(jnp.dot(flat, wdi[...],
                             preferred_element_type=jnp.float32) + bdi[...], 0.)
    h2 = jnp.maximum(jnp.dot(h1, wd0[...],
                             preferred_element_type=jnp.float32) + bd0[...], 0.)
    h3 = jnp.maximum(jnp.dot(h2, wd1[...],
                             preferred_element_type=jnp.float32) + bd1[...], 0.)
    o_ref[...] = jnp.dot(h3, wout[...],
                         preferred_element_type=jnp.float32) + bout[...]


def _tc_block3_pool_head(parts, degp, x, w2, b2, batch3, wdi, bdi, wd0, bd0,
                         wd1, bd1, wout_p, bout_p):
  wspec = pl.BlockSpec((D, D), lambda i: (0, 0))
  bspec = pl.BlockSpec((1, D), lambda i: (0, 0))
  return pl.pallas_call(
      _tc_block3_pool_head_body,
      grid=(GRID,),
      in_specs=[
          pl.BlockSpec((NC, ROW_T, D), lambda i: (0, i, 0)),
          pl.BlockSpec((NW, ROW_T), lambda i: (0, i)),
          pl.BlockSpec((ROW_T, D), lambda i: (i, 0)),
          wspec, bspec,
          pl.BlockSpec((1, 1, ROW_T), lambda i: (i, 0, 0)),
          wspec, bspec, wspec, bspec, wspec, bspec, wspec, bspec,
      ],
      out_specs=pl.BlockSpec((G_POOL, D), lambda i: (0, 0)),
      out_shape=jax.ShapeDtypeStruct((G_POOL, D), jnp.float32),
      scratch_shapes=[
          pltpu.VMEM((G_POOL, D), jnp.float32),
          pltpu.VMEM((G_POOL, D), jnp.float32),
      ],
      compiler_params=pltpu.CompilerParams(
          dimension_semantics=("arbitrary",)),
  )(parts, degp, x, w2, b2, batch3, wdi, bdi, wd0, bd0, wd1, bd1,
    wout_p, bout_p)


def kernel(inputs, edge_index, batch, edge_weight, Win, bin0, W1, b1, W2, b2,
           Wdi, bdi, Wd0, bd0, Wd1, bd1, Wout, bout):
  src = edge_index[0].reshape(NW, NCHUNK, CHUNK)
  dst = edge_index[1].reshape(NW, NCHUNK, CHUNK)
  ew = edge_weight.reshape(NW, NCHUNK, CHUNK)

  parts1, degp = _sc_agg_deg(inputs, src, dst, ew)
  h1 = _tc_block(parts1, degp, None, Win, bin0.reshape(1, D), False)

  (parts2,) = _sc_agg(h1, src, dst, ew)
  h2 = _tc_block(parts2, degp, h1, W1, b1.reshape(1, D), True)

  (parts3,) = _sc_agg(h2, src, dst, ew)

  batch_p = jnp.full((N_PAD,), G_POOL, jnp.int32).at[:N_NODES].set(batch)
  batch3 = batch_p.reshape(GRID, 1, ROW_T)
  wout_p = jnp.zeros((D, D), jnp.float32).at[:, :Wout.shape[1]].set(Wout)
  bout_p = jnp.zeros((1, D), jnp.float32).at[0, :bout.shape[0]].set(bout)
  out = _tc_block3_pool_head(parts3, degp, h2, W2, b2.reshape(1, D), batch3,
                             Wdi, bdi.reshape(1, D), Wd0, bd0.reshape(1, D),
                             Wd1, bd1.reshape(1, D), wout_p, bout_p)
  return out[:, :Wout.shape[1]]
